# Initial kernel scaffold; baseline (speedup 1.0000x reference)
import jax
import jax.numpy as jnp
from jax.experimental import pallas as pl


def kernel(xyz, params):
    raise NotImplementedError("write your pallas kernel here")


# speedup vs baseline: 65.0303x; 65.0303x over previous
import functools

import jax
import jax.numpy as jnp
from jax import lax
from jax.experimental import pallas as pl
from jax.experimental.pallas import tpu as pltpu
from jax.experimental.pallas import tpu_sc as plsc

_B = 8
_K = 64
_EPS = 1e-5
_NW = 32


def _fps_body(npoint, n, xyz_ref, nxyz_ref):
    X = xyz_ref[0]
    Y = xyz_ref[1]
    Z = xyz_ref[2]
    lane = lax.broadcasted_iota(jnp.int32, (_B, n), 1)
    rec_lane = lax.broadcasted_iota(jnp.int32, (_B, npoint), 1)
    nxyz_ref[...] = jnp.zeros((3, _B, npoint), jnp.float32)

    def step(i, carry):
        dist, far = carry
        oh = (lane == far).astype(jnp.float32)
        cx = jnp.sum(X * oh, axis=1, keepdims=True)
        cy = jnp.sum(Y * oh, axis=1, keepdims=True)
        cz = jnp.sum(Z * oh, axis=1, keepdims=True)
        rec = (rec_lane == i).astype(jnp.float32)
        nxyz_ref[0] += cx * rec
        nxyz_ref[1] += cy * rec
        nxyz_ref[2] += cz * rec
        dx = X - cx
        dy = Y - cy
        dz = Z - cz
        d = dx * dx + dy * dy + dz * dz
        dist = jnp.minimum(dist, d)
        m = jnp.max(dist, axis=1, keepdims=True)
        far = jnp.min(jnp.where(dist == m, lane, n), axis=1, keepdims=True)
        return dist, far

    init = (jnp.full((_B, n), 1e10, jnp.float32), jnp.zeros((_B, 1), jnp.int32))
    lax.fori_loop(0, npoint, step, init)


def _fps(xyz3, npoint):
    n = xyz3.shape[2]
    return pl.pallas_call(
        functools.partial(_fps_body, npoint, n),
        out_shape=jax.ShapeDtypeStruct((3, _B, npoint), jnp.float32),
    )(xyz3)


def _bq_body(n, nc, r2, st, x_ref, c_ref, u_ref, t_ref, o_ref):
    b = pl.program_id(0)
    X = x_ref[0, 0][None]
    Y = x_ref[0, 1][None]
    Z = x_ref[0, 2][None]
    ctr = c_ref[0]
    cx = ctr[:, 0:1].reshape(st, 1, 1)
    cy = ctr[:, 1:2].reshape(st, 1, 1)
    cz = ctr[:, 2:3].reshape(st, 1, 1)
    dx = cx - X
    dy = cy - Y
    dz = cz - Z
    d2 = dx * dx + dy * dy + dz * dz
    mask = (d2 <= r2).astype(jnp.float32)
    m2 = mask.reshape(st * nc, 128)
    y = jnp.dot(m2, u_ref[...], preferred_element_type=jnp.float32)
    off = jnp.dot(t_ref[...], y[:, 127:128], preferred_element_type=jnp.float32)
    cnt = jnp.minimum(y + off, float(_K)).reshape(st, nc, 128)
    kl = lax.broadcasted_iota(jnp.int32, (1, _K), 1)
    acc = jnp.zeros((st, _K), jnp.float32)
    for kk in range(_K):
        sk = jnp.sum((cnt > float(kk)).astype(jnp.float32), axis=(1, 2),
                     keepdims=True)
        acc = acc + sk[:, :, 0] * (kl == kk).astype(jnp.float32)
    idxf = float(n) - acc
    first = idxf[:, 0:1]
    idxf = jnp.where(acc == 0.0, first, idxf)
    o_ref[0] = idxf.astype(jnp.int32) + b * n


def _bq(xyz_b3n, centers, radius, st=8):
    n = xyz_b3n.shape[2]
    nc = n // 128
    s = centers.shape[1]
    x4 = xyz_b3n.reshape(_B, 3, nc, 128)
    iu = jnp.arange(128)
    u = (iu[:, None] <= iu[None, :]).astype(jnp.float32)
    rows = st * nc
    ir = jnp.arange(rows)
    t = ((ir[:, None] // nc == ir[None, :] // nc)
         & (ir[None, :] < ir[:, None])).astype(jnp.float32)
    return pl.pallas_call(
        functools.partial(_bq_body, n, nc, radius * radius, st),
        grid=(_B, s // st),
        in_specs=[
            pl.BlockSpec((1, 3, nc, 128), lambda b, i: (b, 0, 0, 0)),
            pl.BlockSpec((1, st, 3), lambda b, i: (b, i, 0)),
            pl.BlockSpec((128, 128), lambda b, i: (0, 0)),
            pl.BlockSpec((rows, rows), lambda b, i: (0, 0)),
        ],
        out_specs=pl.BlockSpec((1, st, _K), lambda b, i: (b, i, 0)),
        out_shape=jax.ShapeDtypeStruct((_B, s, _K), jnp.int32),
    )(x4, centers, u, t)


def _sc_gather(table, idx2, c):
    ng_tot = idx2.shape[0]
    r = ng_tot * 128
    ng = ng_tot // _NW
    mesh = plsc.VectorSubcoreMesh(core_axis_name="c", subcore_axis_name="s")

    @functools.partial(
        pl.kernel,
        mesh=mesh,
        compiler_params=pltpu.CompilerParams(use_tc_tiling_on_sc=False),
        out_type=jax.ShapeDtypeStruct((r, c), jnp.float32),
        scratch_types=[
            pltpu.VMEM((ng, 128), jnp.int32),
            pltpu.VMEM((128, c), jnp.float32),
            pltpu.SemaphoreType.DMA,
        ],
    )
    def gk(table_hbm, idx_hbm, out_hbm, idx_v, rows_v, sem):
        wid = lax.axis_index("s") * 2 + lax.axis_index("c")
        pltpu.sync_copy(idx_hbm.at[pl.ds(wid * ng, ng)], idx_v)

        def body(j, carry):
            pltpu.async_copy(table_hbm.at[idx_v.at[j]], rows_v, sem).wait()
            pltpu.sync_copy(rows_v, out_hbm.at[pl.ds((wid * ng + j) * 128, 128)])
            return carry

        lax.fori_loop(0, ng, body, 0)

    return gk(table, idx2)


def _pass_body(nw, st, c0, c_last, rtot, want_pool, *refs):
    i = pl.program_id(0)
    x0_ref, ctr_ref = refs[0], refs[1]
    w_refs = refs[2:2 + nw]
    b_refs = refs[2 + nw:2 + 2 * nw]
    base = 2 + 2 * nw
    g_refs = refs[base:base + nw - 1]
    be_refs = refs[base + nw - 1:base + 2 * (nw - 1)]
    s_refs = refs[base + 2 * (nw - 1):base + 3 * (nw - 1)]
    outs = refs[base + 3 * (nw - 1):]

    t = st * _K
    x = x0_ref[...].reshape(st, _K, c0) - ctr_ref[...][:, None, :]
    h = x.reshape(t, c0)
    for l in range(nw - 1):
        y = jnp.dot(h, w_refs[l][...], preferred_element_type=jnp.float32) + b_refs[l][...]
        sm = s_refs[l][...]
        mean = sm[0:1, :] / rtot
        var = sm[1:2, :] / rtot - mean * mean
        sc = g_refs[l][...] / jnp.sqrt(var + _EPS)
        tt = be_refs[l][...] - mean * sc
        h = jnp.maximum(y * sc + tt, 0.0)
    y = jnp.dot(h, w_refs[-1][...], preferred_element_type=jnp.float32) + b_refs[-1][...]

    sums_out = outs[0]

    @pl.when(i == 0)
    def _():
        sums_out[...] = jnp.zeros_like(sums_out)

    sums_out[0:1, :] += jnp.sum(y, axis=0, keepdims=True)
    sums_out[1:2, :] += jnp.sum(y * y, axis=0, keepdims=True)
    if want_pool:
        y3 = y.reshape(st, _K, c_last)
        outs[1][...] = jnp.max(y3, axis=1)
        outs[2][...] = jnp.min(y3, axis=1)


def _mlp_pass(x0, ctr, ws, bs, gs, bes, sums_in, st, want_pool):
    rtot, c0 = x0.shape
    nw = len(ws)
    c_last = ws[-1].shape[1]
    t = st * _K
    nsteps = rtot // t
    ins = [x0, ctr] + list(ws) + list(bs) + list(gs) + list(bes) + list(sums_in)
    in_specs = [
        pl.BlockSpec((t, c0), lambda i: (i, 0)),
        pl.BlockSpec((st, c0), lambda i: (i, 0)),
    ]
    for a in ins[2:]:
        in_specs.append(pl.BlockSpec(a.shape, lambda i: tuple(0 for _ in a.shape)))
    out_shape = [jax.ShapeDtypeStruct((2, c_last), jnp.float32)]
    out_specs = [pl.BlockSpec((2, c_last), lambda i: (0, 0))]
    if want_pool:
        srows = rtot // _K
        out_shape += [jax.ShapeDtypeStruct((srows, c_last), jnp.float32)] * 2
        out_specs += [pl.BlockSpec((st, c_last), lambda i: (i, 0))] * 2
    res = pl.pallas_call(
        functools.partial(_pass_body, nw, st, c0, c_last, float(rtot), want_pool),
        grid=(nsteps,),
        in_specs=in_specs,
        out_specs=out_specs,
        out_shape=out_shape,
    )(*ins)
    return res if want_pool else res[0]


def _finalize_body(rtot, pmax_ref, pmin_ref, sums_ref, g_ref, be_ref, o_ref):
    sm = sums_ref[...]
    mean = sm[0:1, :] / rtot
    var = sm[1:2, :] / rtot - mean * mean
    sc = g_ref[...] / jnp.sqrt(var + _EPS)
    tt = be_ref[...] - mean * sc
    sel = jnp.where(sc >= 0.0, pmax_ref[...], pmin_ref[...])
    o_ref[...] = jnp.maximum(sel * sc + tt, 0.0)


def _finalize(pmax, pmin, sums, g, be):
    rtot_pool, c = pmax.shape
    return pl.pallas_call(
        functools.partial(_finalize_body, float(rtot_pool * _K)),
        out_shape=jax.ShapeDtypeStruct((rtot_pool, c), jnp.float32),
    )(pmax, pmin, sums, g, be)


def _sa3dec_body(rows_ref, w1, b1, g1, e1, w2, b2, g2, e2, w3, b3, g3, e3,
                 d1, c1, d2, c2, d3, c3, o_ref):
    h = rows_ref[...]
    for (w, b, g, e) in ((w1, b1, g1, e1), (w2, b2, g2, e2), (w3, b3, g3, e3)):
        y = jnp.dot(h, w[...], preferred_element_type=jnp.float32) + b[...]
        mean = jnp.mean(y, axis=0, keepdims=True)
        dvi = y - mean
        var = jnp.mean(dvi * dvi, axis=0, keepdims=True)
        h = jnp.maximum(dvi / jnp.sqrt(var + _EPS) * g[...] + e[...], 0.0)
    pooled = jnp.max(h.reshape(_B, 128, 1024), axis=1)
    hh = jnp.maximum(jnp.dot(pooled, d1[...], preferred_element_type=jnp.float32) + c1[...], 0.0)
    hh = jnp.maximum(jnp.dot(hh, d2[...], preferred_element_type=jnp.float32) + c2[...], 0.0)
    o_ref[...] = jnp.dot(hh, d3[...], preferred_element_type=jnp.float32) + c3[...]


def _sa3dec(rows3, args):
    return pl.pallas_call(
        _sa3dec_body,
        out_shape=jax.ShapeDtypeStruct((_B, 768), jnp.float32),
    )(rows3, *args)


def _pad_w(w, rows):
    if w.shape[0] == rows:
        return w
    return jnp.concatenate(
        [w, jnp.zeros((rows - w.shape[0], w.shape[1]), jnp.float32)], axis=0)


def _sa_level(x0, ctr, layers, c0, st):
    ws = [_pad_w(layers[0]['W'], c0), layers[1]['W'], layers[2]['W']]
    bs = [l['b'][None, :] for l in layers]
    gs = [l['gamma'][None, :] for l in layers]
    bes = [l['beta'][None, :] for l in layers]
    s1 = _mlp_pass(x0, ctr, ws[:1], bs[:1], [], [], [], st, False)
    s2 = _mlp_pass(x0, ctr, ws[:2], bs[:2], gs[:1], bes[:1], [s1], st, False)
    s3, pmax, pmin = _mlp_pass(x0, ctr, ws, bs, gs[:2], bes[:2], [s1, s2], st, True)
    return _finalize(pmax, pmin, s3, gs[2], bes[2])


def kernel(xyz, params):
    pts = jnp.transpose(xyz, (0, 2, 1))
    b, n1, _ = pts.shape
    xyz3 = jnp.transpose(pts[..., 0:3], (2, 0, 1))

    nxyz1 = _fps(xyz3, 512)
    new1 = jnp.transpose(nxyz1, (1, 2, 0))
    gidx1 = _bq(jnp.transpose(xyz3, (1, 0, 2)), new1, 0.015)
    table1 = jnp.concatenate(
        [pts, jnp.zeros((b, n1, 2), jnp.float32)], axis=-1).reshape(b * n1, 8)
    x01 = _sc_gather(table1, gidx1.reshape(-1, 128), 8)
    ctr1 = jnp.concatenate(
        [new1, jnp.zeros((b, 512, 5), jnp.float32)], axis=-1).reshape(b * 512, 8)
    l1p = _sa_level(x01, ctr1, params['sa1'], 8, 64)

    nxyz2 = _fps(nxyz1, 128)
    new2 = jnp.transpose(nxyz2, (1, 2, 0))
    gidx2 = _bq(jnp.transpose(nxyz1, (1, 0, 2)), new2, 0.04)
    table2 = jnp.concatenate(
        [new1, l1p.reshape(b, 512, 128), jnp.zeros((b, 512, 5), jnp.float32)],
        axis=-1).reshape(b * 512, 136)
    x02 = _sc_gather(table2, gidx2.reshape(-1, 128), 136)
    ctr2 = jnp.concatenate(
        [new2, jnp.zeros((b, 128, 133), jnp.float32)], axis=-1).reshape(b * 128, 136)
    l2p = _sa_level(x02, ctr2, params['sa2'], 136, 64)

    rows3 = jnp.concatenate(
        [new2, l2p.reshape(b, 128, 256), jnp.zeros((b, 128, 5), jnp.float32)],
        axis=-1).reshape(b * 128, 264)
    sa3 = params['sa3']
    dec = params['decoder']
    args = []
    for l, rows in zip(sa3, (264, 256, 256)):
        args += [_pad_w(l['W'], rows), l['b'][None, :],
                 l['gamma'][None, :], l['beta'][None, :]]
    for l in dec:
        args += [l['W'], l['b'][None, :]]
    out8 = _sa3dec(rows3, args)

    return (out8.reshape(b, 256, 3),
            jnp.transpose(new2, (0, 2, 1)),
            jnp.zeros((b, 3, 1), jnp.float32))


# speedup vs baseline: 96.5631x; 1.4849x over previous
import functools

import jax
import jax.numpy as jnp
from jax import lax
from jax.experimental import pallas as pl
from jax.experimental.pallas import tpu as pltpu
from jax.experimental.pallas import tpu_sc as plsc

_B = 8
_K = 64
_EPS = 1e-5
_NW = 32


def _fps_body(npoint, n, xyz_ref, nxyz_ref):
    X = xyz_ref[0]
    Y = xyz_ref[1]
    Z = xyz_ref[2]
    lane = lax.broadcasted_iota(jnp.int32, (_B, n), 1)
    rec_lane = lax.broadcasted_iota(jnp.int32, (_B, npoint), 1)
    nxyz_ref[...] = jnp.zeros((3, _B, npoint), jnp.float32)

    def step(i, carry):
        dist, far = carry
        oh = (lane == far).astype(jnp.float32)
        cx = jnp.sum(X * oh, axis=1, keepdims=True)
        cy = jnp.sum(Y * oh, axis=1, keepdims=True)
        cz = jnp.sum(Z * oh, axis=1, keepdims=True)
        rec = (rec_lane == i).astype(jnp.float32)
        nxyz_ref[0] += cx * rec
        nxyz_ref[1] += cy * rec
        nxyz_ref[2] += cz * rec
        dx = X - cx
        dy = Y - cy
        dz = Z - cz
        d = dx * dx + dy * dy + dz * dz
        dist = jnp.minimum(dist, d)
        m = jnp.max(dist, axis=1, keepdims=True)
        far = jnp.min(jnp.where(dist == m, lane, n), axis=1, keepdims=True)
        return dist, far

    init = (jnp.full((_B, n), 1e10, jnp.float32), jnp.zeros((_B, 1), jnp.int32))
    lax.fori_loop(0, npoint, step, init)


def _fps(xyz3, npoint):
    n = xyz3.shape[2]
    return pl.pallas_call(
        functools.partial(_fps_body, npoint, n),
        out_shape=jax.ShapeDtypeStruct((3, _B, npoint), jnp.float32),
    )(xyz3)


def _bq_body(n, nc, r2, st, x_ref, c_ref, u_ref, t_ref, u2_ref, o_ref):
    b = pl.program_id(0)
    X = x_ref[0, 0][None]
    Y = x_ref[0, 1][None]
    Z = x_ref[0, 2][None]
    ctr = c_ref[0]
    cx = ctr[:, 0:1].reshape(st, 1, 1)
    cy = ctr[:, 1:2].reshape(st, 1, 1)
    cz = ctr[:, 2:3].reshape(st, 1, 1)
    dx = cx - X
    dy = cy - Y
    dz = cz - Z
    d2 = dx * dx + dy * dy + dz * dz
    mask = (d2 <= r2).astype(jnp.float32)
    m2 = mask.reshape(st * nc, 128)
    y = jnp.dot(m2, u_ref[...], preferred_element_type=jnp.float32)
    off = jnp.dot(t_ref[...], y[:, 127:128], preferred_element_type=jnp.float32)
    cnt = (y + off).reshape(st, nc, 128)
    csum = jnp.sum(mask, axis=2)
    hi = jnp.dot(csum, u2_ref[...], preferred_element_type=jnp.float32)
    kcol = lax.broadcasted_iota(jnp.int32, (1, _K, 1), 1).astype(jnp.float32)
    c1 = (hi[:, None, :] <= kcol).astype(jnp.float32)
    nfull = jnp.sum(c1, axis=2, keepdims=True)
    c1prev = jnp.concatenate(
        [jnp.ones((st, _K, 1), jnp.float32), c1[:, :, :nc - 1]], axis=2)
    e = c1prev - c1
    cv = lax.dot_general(e, cnt, (((2,), (1,)), ((0,), (0,))),
                         preferred_element_type=jnp.float32)
    partial = jnp.sum((cv <= kcol).astype(jnp.float32), axis=2, keepdims=True)
    idxf = 128.0 * nfull[:, :, 0] + partial[:, :, 0]
    first = idxf[:, 0:1]
    idxf = jnp.where(idxf >= float(n), first, idxf)
    o_ref[0] = idxf.astype(jnp.int32) + b * n


def _bq(xyz_b3n, centers, radius, st=16):
    n = xyz_b3n.shape[2]
    nc = n // 128
    s = centers.shape[1]
    x4 = xyz_b3n.reshape(_B, 3, nc, 128)
    iu = jnp.arange(128)
    u = (iu[:, None] <= iu[None, :]).astype(jnp.float32)
    rows = st * nc
    ir = jnp.arange(rows)
    t = ((ir[:, None] // nc == ir[None, :] // nc)
         & (ir[None, :] < ir[:, None])).astype(jnp.float32)
    ic = jnp.arange(nc)
    u2 = (ic[:, None] <= ic[None, :]).astype(jnp.float32)
    return pl.pallas_call(
        functools.partial(_bq_body, n, nc, radius * radius, st),
        grid=(_B, s // st),
        in_specs=[
            pl.BlockSpec((1, 3, nc, 128), lambda b, i: (b, 0, 0, 0)),
            pl.BlockSpec((1, st, 3), lambda b, i: (b, i, 0)),
            pl.BlockSpec((128, 128), lambda b, i: (0, 0)),
            pl.BlockSpec((rows, rows), lambda b, i: (0, 0)),
            pl.BlockSpec((nc, nc), lambda b, i: (0, 0)),
        ],
        out_specs=pl.BlockSpec((1, st, _K), lambda b, i: (b, i, 0)),
        out_shape=jax.ShapeDtypeStruct((_B, s, _K), jnp.int32),
    )(x4, centers, u, t, u2)


def _sc_gather(table, idx2, c):
    ng_tot = idx2.shape[0]
    r = ng_tot * 128
    ng = ng_tot // _NW
    mesh = plsc.VectorSubcoreMesh(core_axis_name="c", subcore_axis_name="s")

    @functools.partial(
        pl.kernel,
        mesh=mesh,
        compiler_params=pltpu.CompilerParams(use_tc_tiling_on_sc=False),
        out_type=jax.ShapeDtypeStruct((r, c), jnp.float32),
        scratch_types=[
            pltpu.VMEM((ng, 128), jnp.int32),
            pltpu.VMEM((128, c), jnp.float32),
            pltpu.SemaphoreType.DMA,
        ],
    )
    def gk(table_hbm, idx_hbm, out_hbm, idx_v, rows_v, sem):
        wid = lax.axis_index("s") * 2 + lax.axis_index("c")
        pltpu.sync_copy(idx_hbm.at[pl.ds(wid * ng, ng)], idx_v)

        def body(j, carry):
            pltpu.async_copy(table_hbm.at[idx_v.at[j]], rows_v, sem).wait()
            pltpu.sync_copy(rows_v, out_hbm.at[pl.ds((wid * ng + j) * 128, 128)])
            return carry

        lax.fori_loop(0, ng, body, 0)

    return gk(table, idx2)


def _pass_body(nw, st, c0, c_last, rtot, want_pool, *refs):
    i = pl.program_id(0)
    x0_ref, ctr_ref = refs[0], refs[1]
    w_refs = refs[2:2 + nw]
    b_refs = refs[2 + nw:2 + 2 * nw]
    base = 2 + 2 * nw
    g_refs = refs[base:base + nw - 1]
    be_refs = refs[base + nw - 1:base + 2 * (nw - 1)]
    s_refs = refs[base + 2 * (nw - 1):base + 3 * (nw - 1)]
    outs = refs[base + 3 * (nw - 1):]

    t = st * _K
    x = x0_ref[...].reshape(st, _K, c0) - ctr_ref[...][:, None, :]
    h = x.reshape(t, c0)
    for l in range(nw - 1):
        y = jnp.dot(h, w_refs[l][...], preferred_element_type=jnp.float32) + b_refs[l][...]
        sm = s_refs[l][...]
        mean = sm[0:1, :] / rtot
        var = sm[1:2, :] / rtot - mean * mean
        sc = g_refs[l][...] / jnp.sqrt(var + _EPS)
        tt = be_refs[l][...] - mean * sc
        h = jnp.maximum(y * sc + tt, 0.0)
    y = jnp.dot(h, w_refs[-1][...], preferred_element_type=jnp.float32) + b_refs[-1][...]

    sums_out = outs[0]

    @pl.when(i == 0)
    def _():
        sums_out[...] = jnp.zeros_like(sums_out)

    sums_out[0:1, :] += jnp.sum(y, axis=0, keepdims=True)
    sums_out[1:2, :] += jnp.sum(y * y, axis=0, keepdims=True)
    if want_pool:
        y3 = y.reshape(st, _K, c_last)
        outs[1][...] = jnp.max(y3, axis=1)
        outs[2][...] = jnp.min(y3, axis=1)


def _mlp_pass(x0, ctr, ws, bs, gs, bes, sums_in, st, want_pool):
    rtot, c0 = x0.shape
    nw = len(ws)
    c_last = ws[-1].shape[1]
    t = st * _K
    nsteps = rtot // t
    ins = [x0, ctr] + list(ws) + list(bs) + list(gs) + list(bes) + list(sums_in)
    in_specs = [
        pl.BlockSpec((t, c0), lambda i: (i, 0)),
        pl.BlockSpec((st, c0), lambda i: (i, 0)),
    ]
    for a in ins[2:]:
        in_specs.append(pl.BlockSpec(a.shape, lambda i: tuple(0 for _ in a.shape)))
    out_shape = [jax.ShapeDtypeStruct((2, c_last), jnp.float32)]
    out_specs = [pl.BlockSpec((2, c_last), lambda i: (0, 0))]
    if want_pool:
        srows = rtot // _K
        out_shape += [jax.ShapeDtypeStruct((srows, c_last), jnp.float32)] * 2
        out_specs += [pl.BlockSpec((st, c_last), lambda i: (i, 0))] * 2
    res = pl.pallas_call(
        functools.partial(_pass_body, nw, st, c0, c_last, float(rtot), want_pool),
        grid=(nsteps,),
        in_specs=in_specs,
        out_specs=out_specs,
        out_shape=out_shape,
    )(*ins)
    return res if want_pool else res[0]


def _finalize_body(rtot, pmax_ref, pmin_ref, sums_ref, g_ref, be_ref, o_ref):
    sm = sums_ref[...]
    mean = sm[0:1, :] / rtot
    var = sm[1:2, :] / rtot - mean * mean
    sc = g_ref[...] / jnp.sqrt(var + _EPS)
    tt = be_ref[...] - mean * sc
    sel = jnp.where(sc >= 0.0, pmax_ref[...], pmin_ref[...])
    o_ref[...] = jnp.maximum(sel * sc + tt, 0.0)


def _finalize(pmax, pmin, sums, g, be):
    rtot_pool, c = pmax.shape
    return pl.pallas_call(
        functools.partial(_finalize_body, float(rtot_pool * _K)),
        out_shape=jax.ShapeDtypeStruct((rtot_pool, c), jnp.float32),
    )(pmax, pmin, sums, g, be)


def _sa3dec_body(rows_ref, w1, b1, g1, e1, w2, b2, g2, e2, w3, b3, g3, e3,
                 d1, c1, d2, c2, d3, c3, o_ref):
    h = rows_ref[...]
    for (w, b, g, e) in ((w1, b1, g1, e1), (w2, b2, g2, e2), (w3, b3, g3, e3)):
        y = jnp.dot(h, w[...], preferred_element_type=jnp.float32) + b[...]
        mean = jnp.mean(y, axis=0, keepdims=True)
        dvi = y - mean
        var = jnp.mean(dvi * dvi, axis=0, keepdims=True)
        h = jnp.maximum(dvi / jnp.sqrt(var + _EPS) * g[...] + e[...], 0.0)
    pooled = jnp.max(h.reshape(_B, 128, 1024), axis=1)
    hh = jnp.maximum(jnp.dot(pooled, d1[...], preferred_element_type=jnp.float32) + c1[...], 0.0)
    hh = jnp.maximum(jnp.dot(hh, d2[...], preferred_element_type=jnp.float32) + c2[...], 0.0)
    o_ref[...] = jnp.dot(hh, d3[...], preferred_element_type=jnp.float32) + c3[...]


def _sa3dec(rows3, args):
    return pl.pallas_call(
        _sa3dec_body,
        out_shape=jax.ShapeDtypeStruct((_B, 768), jnp.float32),
    )(rows3, *args)


def _pad_w(w, rows):
    if w.shape[0] == rows:
        return w
    return jnp.concatenate(
        [w, jnp.zeros((rows - w.shape[0], w.shape[1]), jnp.float32)], axis=0)


def _sa_level(x0, ctr, layers, c0, st):
    ws = [_pad_w(layers[0]['W'], c0), layers[1]['W'], layers[2]['W']]
    bs = [l['b'][None, :] for l in layers]
    gs = [l['gamma'][None, :] for l in layers]
    bes = [l['beta'][None, :] for l in layers]
    s1 = _mlp_pass(x0, ctr, ws[:1], bs[:1], [], [], [], st, False)
    s2 = _mlp_pass(x0, ctr, ws[:2], bs[:2], gs[:1], bes[:1], [s1], st, False)
    s3, pmax, pmin = _mlp_pass(x0, ctr, ws, bs, gs[:2], bes[:2], [s1, s2], st, True)
    return _finalize(pmax, pmin, s3, gs[2], bes[2])


def kernel(xyz, params):
    pts = jnp.transpose(xyz, (0, 2, 1))
    b, n1, _ = pts.shape
    xyz3 = jnp.transpose(pts[..., 0:3], (2, 0, 1))

    nxyz1 = _fps(xyz3, 512)
    new1 = jnp.transpose(nxyz1, (1, 2, 0))
    gidx1 = _bq(jnp.transpose(xyz3, (1, 0, 2)), new1, 0.015)
    table1 = jnp.concatenate(
        [pts, jnp.zeros((b, n1, 2), jnp.float32)], axis=-1).reshape(b * n1, 8)
    x01 = _sc_gather(table1, gidx1.reshape(-1, 128), 8)
    ctr1 = jnp.concatenate(
        [new1, jnp.zeros((b, 512, 5), jnp.float32)], axis=-1).reshape(b * 512, 8)
    l1p = _sa_level(x01, ctr1, params['sa1'], 8, 64)

    nxyz2 = _fps(nxyz1, 128)
    new2 = jnp.transpose(nxyz2, (1, 2, 0))
    gidx2 = _bq(jnp.transpose(nxyz1, (1, 0, 2)), new2, 0.04)
    table2 = jnp.concatenate(
        [new1, l1p.reshape(b, 512, 128), jnp.zeros((b, 512, 5), jnp.float32)],
        axis=-1).reshape(b * 512, 136)
    x02 = _sc_gather(table2, gidx2.reshape(-1, 128), 136)
    ctr2 = jnp.concatenate(
        [new2, jnp.zeros((b, 128, 133), jnp.float32)], axis=-1).reshape(b * 128, 136)
    l2p = _sa_level(x02, ctr2, params['sa2'], 136, 64)

    rows3 = jnp.concatenate(
        [new2, l2p.reshape(b, 128, 256), jnp.zeros((b, 128, 5), jnp.float32)],
        axis=-1).reshape(b * 128, 264)
    sa3 = params['sa3']
    dec = params['decoder']
    args = []
    for l, rows in zip(sa3, (264, 256, 256)):
        args += [_pad_w(l['W'], rows), l['b'][None, :],
                 l['gamma'][None, :], l['beta'][None, :]]
    for l in dec:
        args += [l['W'], l['b'][None, :]]
    out8 = _sa3dec(rows3, args)

    return (out8.reshape(b, 256, 3),
            jnp.transpose(new2, (0, 2, 1)),
            jnp.zeros((b, 3, 1), jnp.float32))


# speedup vs baseline: 98.0830x; 1.0157x over previous
import functools

import jax
import jax.numpy as jnp
from jax import lax
from jax.experimental import pallas as pl
from jax.experimental.pallas import tpu as pltpu
from jax.experimental.pallas import tpu_sc as plsc

_B = 8
_K = 64
_EPS = 1e-5
_NW = 32


def _fps_body(npoint, n, xyz_ref, nxyz_ref):
    X = xyz_ref[0]
    Y = xyz_ref[1]
    Z = xyz_ref[2]
    lane = lax.broadcasted_iota(jnp.int32, (_B, n), 1)
    rec_lane = lax.broadcasted_iota(jnp.int32, (_B, npoint), 1)
    nxyz_ref[...] = jnp.zeros((3, _B, npoint), jnp.float32)

    def step(i, carry):
        dist, far = carry
        oh = (lane == far).astype(jnp.float32)
        cx = jnp.sum(X * oh, axis=1, keepdims=True)
        cy = jnp.sum(Y * oh, axis=1, keepdims=True)
        cz = jnp.sum(Z * oh, axis=1, keepdims=True)
        rec = (rec_lane == i).astype(jnp.float32)
        nxyz_ref[0] += cx * rec
        nxyz_ref[1] += cy * rec
        nxyz_ref[2] += cz * rec
        dx = X - cx
        dy = Y - cy
        dz = Z - cz
        d = dx * dx + dy * dy + dz * dz
        dist = jnp.minimum(dist, d)
        m = jnp.max(dist, axis=1, keepdims=True)
        far = jnp.min(jnp.where(dist == m, lane, n), axis=1, keepdims=True)
        return dist, far

    init = (jnp.full((_B, n), 1e10, jnp.float32), jnp.zeros((_B, 1), jnp.int32))
    lax.fori_loop(0, npoint, step, init)


def _fps(xyz3, npoint):
    n = xyz3.shape[2]
    return pl.pallas_call(
        functools.partial(_fps_body, npoint, n),
        out_shape=jax.ShapeDtypeStruct((3, _B, npoint), jnp.float32),
    )(xyz3)


def _bq_body(n, nc, r2, st, x_ref, c_ref, u_ref, t_ref, u2_ref, o_ref):
    b = pl.program_id(0)
    X = x_ref[0, 0][None]
    Y = x_ref[0, 1][None]
    Z = x_ref[0, 2][None]
    ctr = c_ref[0]
    cx = ctr[:, 0:1].reshape(st, 1, 1)
    cy = ctr[:, 1:2].reshape(st, 1, 1)
    cz = ctr[:, 2:3].reshape(st, 1, 1)
    dx = cx - X
    dy = cy - Y
    dz = cz - Z
    d2 = dx * dx + dy * dy + dz * dz
    mask = (d2 <= r2).astype(jnp.float32)
    m2 = mask.reshape(st * nc, 128)
    y = jnp.dot(m2, u_ref[...], preferred_element_type=jnp.float32)
    off = jnp.dot(t_ref[...], y[:, 127:128], preferred_element_type=jnp.float32)
    cnt = (y + off).reshape(st, nc, 128)
    csum = jnp.sum(mask, axis=2)
    hi = jnp.dot(csum, u2_ref[...], preferred_element_type=jnp.float32)
    kcol = lax.broadcasted_iota(jnp.int32, (1, _K, 1), 1).astype(jnp.float32)
    c1 = (hi[:, None, :] <= kcol).astype(jnp.float32)
    nfull = jnp.sum(c1, axis=2, keepdims=True)
    c1prev = jnp.concatenate(
        [jnp.ones((st, _K, 1), jnp.float32), c1[:, :, :nc - 1]], axis=2)
    e = c1prev - c1
    cv = lax.dot_general(e, cnt, (((2,), (1,)), ((0,), (0,))),
                         preferred_element_type=jnp.float32)
    partial = jnp.sum((cv <= kcol).astype(jnp.float32), axis=2, keepdims=True)
    idxf = 128.0 * nfull[:, :, 0] + partial[:, :, 0]
    first = idxf[:, 0:1]
    idxf = jnp.where(idxf >= float(n), first, idxf)
    o_ref[0] = idxf.astype(jnp.int32) + b * n


def _bq(xyz_b3n, centers, radius, st=32):
    n = xyz_b3n.shape[2]
    nc = n // 128
    s = centers.shape[1]
    x4 = xyz_b3n.reshape(_B, 3, nc, 128)
    iu = jnp.arange(128)
    u = (iu[:, None] <= iu[None, :]).astype(jnp.float32)
    rows = st * nc
    ir = jnp.arange(rows)
    t = ((ir[:, None] // nc == ir[None, :] // nc)
         & (ir[None, :] < ir[:, None])).astype(jnp.float32)
    ic = jnp.arange(nc)
    u2 = (ic[:, None] <= ic[None, :]).astype(jnp.float32)
    return pl.pallas_call(
        functools.partial(_bq_body, n, nc, radius * radius, st),
        grid=(_B, s // st),
        in_specs=[
            pl.BlockSpec((1, 3, nc, 128), lambda b, i: (b, 0, 0, 0)),
            pl.BlockSpec((1, st, 3), lambda b, i: (b, i, 0)),
            pl.BlockSpec((128, 128), lambda b, i: (0, 0)),
            pl.BlockSpec((rows, rows), lambda b, i: (0, 0)),
            pl.BlockSpec((nc, nc), lambda b, i: (0, 0)),
        ],
        out_specs=pl.BlockSpec((1, st, _K), lambda b, i: (b, i, 0)),
        out_shape=jax.ShapeDtypeStruct((_B, s, _K), jnp.int32),
    )(x4, centers, u, t, u2)


def _sc_gather(table, idx2, c):
    ng_tot = idx2.shape[0]
    r = ng_tot * 128
    ng = ng_tot // _NW
    mesh = plsc.VectorSubcoreMesh(core_axis_name="c", subcore_axis_name="s")

    @functools.partial(
        pl.kernel,
        mesh=mesh,
        compiler_params=pltpu.CompilerParams(use_tc_tiling_on_sc=False),
        out_type=jax.ShapeDtypeStruct((r, c), jnp.float32),
        scratch_types=[
            pltpu.VMEM((ng, 128), jnp.int32),
            pltpu.VMEM((128, c), jnp.float32),
            pltpu.SemaphoreType.DMA,
        ],
    )
    def gk(table_hbm, idx_hbm, out_hbm, idx_v, rows_v, sem):
        wid = lax.axis_index("s") * 2 + lax.axis_index("c")
        pltpu.sync_copy(idx_hbm.at[pl.ds(wid * ng, ng)], idx_v)

        def body(j, carry):
            pltpu.async_copy(table_hbm.at[idx_v.at[j]], rows_v, sem).wait()
            pltpu.sync_copy(rows_v, out_hbm.at[pl.ds((wid * ng + j) * 128, 128)])
            return carry

        lax.fori_loop(0, ng, body, 0)

    return gk(table, idx2)


def _pass_body(nw, st, c0, c_last, rtot, want_pool, *refs):
    i = pl.program_id(0)
    x0_ref, ctr_ref = refs[0], refs[1]
    w_refs = refs[2:2 + nw]
    b_refs = refs[2 + nw:2 + 2 * nw]
    base = 2 + 2 * nw
    g_refs = refs[base:base + nw - 1]
    be_refs = refs[base + nw - 1:base + 2 * (nw - 1)]
    s_refs = refs[base + 2 * (nw - 1):base + 3 * (nw - 1)]
    outs = refs[base + 3 * (nw - 1):]

    t = st * _K
    x = x0_ref[...].reshape(st, _K, c0) - ctr_ref[...][:, None, :]
    h = x.reshape(t, c0)
    for l in range(nw - 1):
        y = jnp.dot(h, w_refs[l][...], preferred_element_type=jnp.float32) + b_refs[l][...]
        sm = s_refs[l][...]
        mean = sm[0:1, :] / rtot
        var = sm[1:2, :] / rtot - mean * mean
        sc = g_refs[l][...] / jnp.sqrt(var + _EPS)
        tt = be_refs[l][...] - mean * sc
        h = jnp.maximum(y * sc + tt, 0.0)
    y = jnp.dot(h, w_refs[-1][...], preferred_element_type=jnp.float32) + b_refs[-1][...]

    sums_out = outs[0]

    @pl.when(i == 0)
    def _():
        sums_out[...] = jnp.zeros_like(sums_out)

    sums_out[0:1, :] += jnp.sum(y, axis=0, keepdims=True)
    sums_out[1:2, :] += jnp.sum(y * y, axis=0, keepdims=True)
    if want_pool:
        y3 = y.reshape(st, _K, c_last)
        outs[1][...] = jnp.max(y3, axis=1)
        outs[2][...] = jnp.min(y3, axis=1)


def _mlp_pass(x0, ctr, ws, bs, gs, bes, sums_in, st, want_pool):
    rtot, c0 = x0.shape
    nw = len(ws)
    c_last = ws[-1].shape[1]
    t = st * _K
    nsteps = rtot // t
    ins = [x0, ctr] + list(ws) + list(bs) + list(gs) + list(bes) + list(sums_in)
    in_specs = [
        pl.BlockSpec((t, c0), lambda i: (i, 0)),
        pl.BlockSpec((st, c0), lambda i: (i, 0)),
    ]
    for a in ins[2:]:
        in_specs.append(pl.BlockSpec(a.shape, lambda i: tuple(0 for _ in a.shape)))
    out_shape = [jax.ShapeDtypeStruct((2, c_last), jnp.float32)]
    out_specs = [pl.BlockSpec((2, c_last), lambda i: (0, 0))]
    if want_pool:
        srows = rtot // _K
        out_shape += [jax.ShapeDtypeStruct((srows, c_last), jnp.float32)] * 2
        out_specs += [pl.BlockSpec((st, c_last), lambda i: (i, 0))] * 2
    res = pl.pallas_call(
        functools.partial(_pass_body, nw, st, c0, c_last, float(rtot), want_pool),
        grid=(nsteps,),
        in_specs=in_specs,
        out_specs=out_specs,
        out_shape=out_shape,
    )(*ins)
    return res if want_pool else res[0]


def _finalize_body(rtot, pmax_ref, pmin_ref, sums_ref, g_ref, be_ref, o_ref):
    sm = sums_ref[...]
    mean = sm[0:1, :] / rtot
    var = sm[1:2, :] / rtot - mean * mean
    sc = g_ref[...] / jnp.sqrt(var + _EPS)
    tt = be_ref[...] - mean * sc
    sel = jnp.where(sc >= 0.0, pmax_ref[...], pmin_ref[...])
    o_ref[...] = jnp.maximum(sel * sc + tt, 0.0)


def _finalize(pmax, pmin, sums, g, be):
    rtot_pool, c = pmax.shape
    return pl.pallas_call(
        functools.partial(_finalize_body, float(rtot_pool * _K)),
        out_shape=jax.ShapeDtypeStruct((rtot_pool, c), jnp.float32),
    )(pmax, pmin, sums, g, be)


def _sa3dec_body(rows_ref, w1, b1, g1, e1, w2, b2, g2, e2, w3, b3, g3, e3,
                 d1, c1, d2, c2, d3, c3, o_ref):
    h = rows_ref[...]
    for (w, b, g, e) in ((w1, b1, g1, e1), (w2, b2, g2, e2), (w3, b3, g3, e3)):
        y = jnp.dot(h, w[...], preferred_element_type=jnp.float32) + b[...]
        mean = jnp.mean(y, axis=0, keepdims=True)
        dvi = y - mean
        var = jnp.mean(dvi * dvi, axis=0, keepdims=True)
        h = jnp.maximum(dvi / jnp.sqrt(var + _EPS) * g[...] + e[...], 0.0)
    pooled = jnp.max(h.reshape(_B, 128, 1024), axis=1)
    hh = jnp.maximum(jnp.dot(pooled, d1[...], preferred_element_type=jnp.float32) + c1[...], 0.0)
    hh = jnp.maximum(jnp.dot(hh, d2[...], preferred_element_type=jnp.float32) + c2[...], 0.0)
    o_ref[...] = jnp.dot(hh, d3[...], preferred_element_type=jnp.float32) + c3[...]


def _sa3dec(rows3, args):
    return pl.pallas_call(
        _sa3dec_body,
        out_shape=jax.ShapeDtypeStruct((_B, 768), jnp.float32),
    )(rows3, *args)


def _pad_w(w, rows):
    if w.shape[0] == rows:
        return w
    return jnp.concatenate(
        [w, jnp.zeros((rows - w.shape[0], w.shape[1]), jnp.float32)], axis=0)


def _sa_level(x0, ctr, layers, c0, st):
    ws = [_pad_w(layers[0]['W'], c0), layers[1]['W'], layers[2]['W']]
    bs = [l['b'][None, :] for l in layers]
    gs = [l['gamma'][None, :] for l in layers]
    bes = [l['beta'][None, :] for l in layers]
    s1 = _mlp_pass(x0, ctr, ws[:1], bs[:1], [], [], [], st, False)
    s2 = _mlp_pass(x0, ctr, ws[:2], bs[:2], gs[:1], bes[:1], [s1], st, False)
    s3, pmax, pmin = _mlp_pass(x0, ctr, ws, bs, gs[:2], bes[:2], [s1, s2], st, True)
    return _finalize(pmax, pmin, s3, gs[2], bes[2])


def kernel(xyz, params):
    pts = jnp.transpose(xyz, (0, 2, 1))
    b, n1, _ = pts.shape
    xyz3 = jnp.transpose(pts[..., 0:3], (2, 0, 1))

    nxyz1 = _fps(xyz3, 512)
    new1 = jnp.transpose(nxyz1, (1, 2, 0))
    gidx1 = _bq(jnp.transpose(xyz3, (1, 0, 2)), new1, 0.015)
    table1 = jnp.concatenate(
        [pts, jnp.zeros((b, n1, 2), jnp.float32)], axis=-1).reshape(b * n1, 8)
    x01 = _sc_gather(table1, gidx1.reshape(-1, 128), 8)
    ctr1 = jnp.concatenate(
        [new1, jnp.zeros((b, 512, 5), jnp.float32)], axis=-1).reshape(b * 512, 8)
    l1p = _sa_level(x01, ctr1, params['sa1'], 8, 64)

    nxyz2 = _fps(nxyz1, 128)
    new2 = jnp.transpose(nxyz2, (1, 2, 0))
    gidx2 = _bq(jnp.transpose(nxyz1, (1, 0, 2)), new2, 0.04)
    table2 = jnp.concatenate(
        [new1, l1p.reshape(b, 512, 128), jnp.zeros((b, 512, 5), jnp.float32)],
        axis=-1).reshape(b * 512, 136)
    x02 = _sc_gather(table2, gidx2.reshape(-1, 128), 136)
    ctr2 = jnp.concatenate(
        [new2, jnp.zeros((b, 128, 133), jnp.float32)], axis=-1).reshape(b * 128, 136)
    l2p = _sa_level(x02, ctr2, params['sa2'], 136, 64)

    rows3 = jnp.concatenate(
        [new2, l2p.reshape(b, 128, 256), jnp.zeros((b, 128, 5), jnp.float32)],
        axis=-1).reshape(b * 128, 264)
    sa3 = params['sa3']
    dec = params['decoder']
    args = []
    for l, rows in zip(sa3, (264, 256, 256)):
        args += [_pad_w(l['W'], rows), l['b'][None, :],
                 l['gamma'][None, :], l['beta'][None, :]]
    for l in dec:
        args += [l['W'], l['b'][None, :]]
    out8 = _sa3dec(rows3, args)

    return (out8.reshape(b, 256, 3),
            jnp.transpose(new2, (0, 2, 1)),
            jnp.zeros((b, 3, 1), jnp.float32))


# speedup vs baseline: 108.8278x; 1.1095x over previous
import functools

import numpy as np

import jax
import jax.numpy as jnp
from jax import lax
from jax.experimental import pallas as pl
from jax.experimental.pallas import tpu as pltpu
from jax.experimental.pallas import tpu_sc as plsc

_B = 8
_K = 64
_EPS = 1e-5
_NW = 32


def _fps_body(npoint, n, xyz_ref, nxyz_ref):
    X = xyz_ref[0]
    Y = xyz_ref[1]
    Z = xyz_ref[2]
    lane = lax.broadcasted_iota(jnp.int32, (_B, n), 1)
    rec_lane = lax.broadcasted_iota(jnp.int32, (_B, npoint), 1)
    nxyz_ref[...] = jnp.zeros((3, _B, npoint), jnp.float32)

    def step(i, carry):
        dist, far = carry
        oh = (lane == far).astype(jnp.float32)
        cx = jnp.sum(X * oh, axis=1, keepdims=True)
        cy = jnp.sum(Y * oh, axis=1, keepdims=True)
        cz = jnp.sum(Z * oh, axis=1, keepdims=True)
        rec = (rec_lane == i).astype(jnp.float32)
        nxyz_ref[0] += cx * rec
        nxyz_ref[1] += cy * rec
        nxyz_ref[2] += cz * rec
        dx = X - cx
        dy = Y - cy
        dz = Z - cz
        d = dx * dx + dy * dy + dz * dz
        dist = jnp.minimum(dist, d)
        m = jnp.max(dist, axis=1, keepdims=True)
        far = jnp.min(jnp.where(dist == m, lane, n), axis=1, keepdims=True)
        return dist, far

    init = (jnp.full((_B, n), 1e10, jnp.float32), jnp.zeros((_B, 1), jnp.int32))
    lax.fori_loop(0, npoint, step, init)


def _fps(xyz3, npoint):
    n = xyz3.shape[2]
    return pl.pallas_call(
        functools.partial(_fps_body, npoint, n),
        out_shape=jax.ShapeDtypeStruct((3, _B, npoint), jnp.float32),
    )(xyz3)


def _bq_body(n, nc, r2, st, x_ref, c_ref, u_ref, u2_ref, o_ref):
    b = pl.program_id(0)
    X = x_ref[0, 0][None]
    Y = x_ref[0, 1][None]
    Z = x_ref[0, 2][None]
    ctr = c_ref[0]
    cx = ctr[:, 0:1].reshape(st, 1, 1)
    cy = ctr[:, 1:2].reshape(st, 1, 1)
    cz = ctr[:, 2:3].reshape(st, 1, 1)
    dx = cx - X
    dy = cy - Y
    dz = cz - Z
    d2 = dx * dx + dy * dy + dz * dz
    mask = (d2 <= r2).astype(jnp.float32)
    m2 = mask.reshape(st * nc, 128)
    y = jnp.dot(m2, u_ref[...], preferred_element_type=jnp.float32)
    csum = jnp.sum(mask, axis=2)
    hi = jnp.dot(csum, u2_ref[...], preferred_element_type=jnp.float32)
    cnt = y.reshape(st, nc, 128)
    kcol = lax.broadcasted_iota(jnp.int32, (1, _K, 1), 1).astype(jnp.float32)
    c1 = (hi[:, None, :] <= kcol).astype(jnp.float32)
    nfull = jnp.sum(c1, axis=2, keepdims=True)
    c1prev = jnp.concatenate(
        [jnp.ones((st, _K, 1), jnp.float32), c1[:, :, :nc - 1]], axis=2)
    e = c1prev - c1
    cv = lax.dot_general(e, cnt, (((2,), (1,)), ((0,), (0,))),
                         preferred_element_type=jnp.float32)
    ex = jnp.sum(e * (hi - csum)[:, None, :], axis=2, keepdims=True)
    partial = jnp.sum((cv + ex <= kcol).astype(jnp.float32), axis=2, keepdims=True)
    idxf = 128.0 * nfull[:, :, 0] + partial[:, :, 0]
    first = idxf[:, 0:1]
    idxf = jnp.where(idxf >= float(n), first, idxf)
    o_ref[0] = idxf.astype(jnp.int32) + b * n


def _bq(xyz_b3n, centers, radius, st=32):
    n = xyz_b3n.shape[2]
    nc = n // 128
    s = centers.shape[1]
    x4 = xyz_b3n.reshape(_B, 3, nc, 128)
    iu = np.arange(128)
    u = jnp.asarray((iu[:, None] <= iu[None, :]), jnp.float32)
    ic = np.arange(nc)
    u2 = jnp.asarray((ic[:, None] <= ic[None, :]), jnp.float32)
    return pl.pallas_call(
        functools.partial(_bq_body, n, nc, radius * radius, st),
        grid=(_B, s // st),
        in_specs=[
            pl.BlockSpec((1, 3, nc, 128), lambda b, i: (b, 0, 0, 0)),
            pl.BlockSpec((1, st, 3), lambda b, i: (b, i, 0)),
            pl.BlockSpec((128, 128), lambda b, i: (0, 0)),
            pl.BlockSpec((nc, nc), lambda b, i: (0, 0)),
        ],
        out_specs=pl.BlockSpec((1, st, _K), lambda b, i: (b, i, 0)),
        out_shape=jax.ShapeDtypeStruct((_B, s, _K), jnp.int32),
    )(x4, centers, u, u2)


def _sc_gather(table, idx2, c):
    ng_tot = idx2.shape[0]
    r = ng_tot * 128
    ng = ng_tot // _NW
    mesh = plsc.VectorSubcoreMesh(core_axis_name="c", subcore_axis_name="s")

    @functools.partial(
        pl.kernel,
        mesh=mesh,
        compiler_params=pltpu.CompilerParams(use_tc_tiling_on_sc=False),
        out_type=jax.ShapeDtypeStruct((r, c), jnp.float32),
        scratch_types=[
            pltpu.VMEM((ng, 128), jnp.int32),
            pltpu.VMEM((128, c), jnp.float32),
            pltpu.SemaphoreType.DMA,
        ],
    )
    def gk(table_hbm, idx_hbm, out_hbm, idx_v, rows_v, sem):
        wid = lax.axis_index("s") * 2 + lax.axis_index("c")
        pltpu.sync_copy(idx_hbm.at[pl.ds(wid * ng, ng)], idx_v)

        def body(j, carry):
            pltpu.async_copy(table_hbm.at[idx_v.at[j]], rows_v, sem).wait()
            pltpu.sync_copy(rows_v, out_hbm.at[pl.ds((wid * ng + j) * 128, 128)])
            return carry

        lax.fori_loop(0, ng, body, 0)

    return gk(table, idx2)


def _pass_body(nw, st, c0, c_last, rtot, want_pool, *refs):
    i = pl.program_id(0)
    x0_ref, ctr_ref = refs[0], refs[1]
    w_refs = refs[2:2 + nw]
    b_refs = refs[2 + nw:2 + 2 * nw]
    base = 2 + 2 * nw
    g_refs = refs[base:base + nw - 1]
    be_refs = refs[base + nw - 1:base + 2 * (nw - 1)]
    s_refs = refs[base + 2 * (nw - 1):base + 3 * (nw - 1)]
    outs = refs[base + 3 * (nw - 1):]

    t = st * _K
    x = x0_ref[...].reshape(st, _K, c0) - ctr_ref[...][:, None, :]
    h = x.reshape(t, c0)
    for l in range(nw - 1):
        y = jnp.dot(h, w_refs[l][...], preferred_element_type=jnp.float32) + b_refs[l][...]
        sm = s_refs[l][...]
        mean = sm[0:1, :] / rtot
        var = sm[1:2, :] / rtot - mean * mean
        sc = g_refs[l][...] / jnp.sqrt(var + _EPS)
        tt = be_refs[l][...] - mean * sc
        h = jnp.maximum(y * sc + tt, 0.0)
    y = jnp.dot(h, w_refs[-1][...], preferred_element_type=jnp.float32) + b_refs[-1][...]

    sums_out = outs[0]

    @pl.when(i == 0)
    def _():
        sums_out[...] = jnp.zeros_like(sums_out)

    sums_out[0:1, :] += jnp.sum(y, axis=0, keepdims=True)
    sums_out[1:2, :] += jnp.sum(y * y, axis=0, keepdims=True)
    if want_pool:
        y3 = y.reshape(st, _K, c_last)
        outs[1][...] = jnp.max(y3, axis=1)
        outs[2][...] = jnp.min(y3, axis=1)


def _mlp_pass(x0, ctr, ws, bs, gs, bes, sums_in, st, want_pool):
    rtot, c0 = x0.shape
    nw = len(ws)
    c_last = ws[-1].shape[1]
    t = st * _K
    nsteps = rtot // t
    ins = [x0, ctr] + list(ws) + list(bs) + list(gs) + list(bes) + list(sums_in)
    in_specs = [
        pl.BlockSpec((t, c0), lambda i: (i, 0)),
        pl.BlockSpec((st, c0), lambda i: (i, 0)),
    ]
    for a in ins[2:]:
        in_specs.append(pl.BlockSpec(a.shape, lambda i: tuple(0 for _ in a.shape)))
    out_shape = [jax.ShapeDtypeStruct((2, c_last), jnp.float32)]
    out_specs = [pl.BlockSpec((2, c_last), lambda i: (0, 0))]
    if want_pool:
        srows = rtot // _K
        out_shape += [jax.ShapeDtypeStruct((srows, c_last), jnp.float32)] * 2
        out_specs += [pl.BlockSpec((st, c_last), lambda i: (i, 0))] * 2
    res = pl.pallas_call(
        functools.partial(_pass_body, nw, st, c0, c_last, float(rtot), want_pool),
        grid=(nsteps,),
        in_specs=in_specs,
        out_specs=out_specs,
        out_shape=out_shape,
    )(*ins)
    return res if want_pool else res[0]


def _finalize_body(rtot, pmax_ref, pmin_ref, sums_ref, g_ref, be_ref, o_ref):
    sm = sums_ref[...]
    mean = sm[0:1, :] / rtot
    var = sm[1:2, :] / rtot - mean * mean
    sc = g_ref[...] / jnp.sqrt(var + _EPS)
    tt = be_ref[...] - mean * sc
    sel = jnp.where(sc >= 0.0, pmax_ref[...], pmin_ref[...])
    o_ref[...] = jnp.maximum(sel * sc + tt, 0.0)


def _finalize(pmax, pmin, sums, g, be):
    rtot_pool, c = pmax.shape
    return pl.pallas_call(
        functools.partial(_finalize_body, float(rtot_pool * _K)),
        out_shape=jax.ShapeDtypeStruct((rtot_pool, c), jnp.float32),
    )(pmax, pmin, sums, g, be)


def _sa3dec_body(rows_ref, w1, b1, g1, e1, w2, b2, g2, e2, w3, b3, g3, e3,
                 d1, c1, d2, c2, d3, c3, o_ref):
    h = rows_ref[...]
    for (w, b, g, e) in ((w1, b1, g1, e1), (w2, b2, g2, e2), (w3, b3, g3, e3)):
        y = jnp.dot(h, w[...], preferred_element_type=jnp.float32) + b[...]
        mean = jnp.mean(y, axis=0, keepdims=True)
        dvi = y - mean
        var = jnp.mean(dvi * dvi, axis=0, keepdims=True)
        h = jnp.maximum(dvi / jnp.sqrt(var + _EPS) * g[...] + e[...], 0.0)
    pooled = jnp.max(h.reshape(_B, 128, 1024), axis=1)
    hh = jnp.maximum(jnp.dot(pooled, d1[...], preferred_element_type=jnp.float32) + c1[...], 0.0)
    hh = jnp.maximum(jnp.dot(hh, d2[...], preferred_element_type=jnp.float32) + c2[...], 0.0)
    o_ref[...] = jnp.dot(hh, d3[...], preferred_element_type=jnp.float32) + c3[...]


def _sa3dec(rows3, args):
    return pl.pallas_call(
        _sa3dec_body,
        out_shape=jax.ShapeDtypeStruct((_B, 768), jnp.float32),
    )(rows3, *args)


def _pad_w(w, rows):
    if w.shape[0] == rows:
        return w
    return jnp.concatenate(
        [w, jnp.zeros((rows - w.shape[0], w.shape[1]), jnp.float32)], axis=0)


def _sa_level(x0, ctr, layers, c0, st):
    ws = [_pad_w(layers[0]['W'], c0), layers[1]['W'], layers[2]['W']]
    bs = [l['b'][None, :] for l in layers]
    gs = [l['gamma'][None, :] for l in layers]
    bes = [l['beta'][None, :] for l in layers]
    s1 = _mlp_pass(x0, ctr, ws[:1], bs[:1], [], [], [], st, False)
    s2 = _mlp_pass(x0, ctr, ws[:2], bs[:2], gs[:1], bes[:1], [s1], st, False)
    s3, pmax, pmin = _mlp_pass(x0, ctr, ws, bs, gs[:2], bes[:2], [s1, s2], st, True)
    return _finalize(pmax, pmin, s3, gs[2], bes[2])


def kernel(xyz, params):
    pts = jnp.transpose(xyz, (0, 2, 1))
    b, n1, _ = pts.shape
    xyz3 = jnp.transpose(pts[..., 0:3], (2, 0, 1))

    nxyz1 = _fps(xyz3, 512)
    new1 = jnp.transpose(nxyz1, (1, 2, 0))
    gidx1 = _bq(jnp.transpose(xyz3, (1, 0, 2)), new1, 0.015)
    table1 = jnp.concatenate(
        [pts, jnp.zeros((b, n1, 2), jnp.float32)], axis=-1).reshape(b * n1, 8)
    x01 = _sc_gather(table1, gidx1.reshape(-1, 128), 8)
    ctr1 = jnp.concatenate(
        [new1, jnp.zeros((b, 512, 5), jnp.float32)], axis=-1).reshape(b * 512, 8)
    l1p = _sa_level(x01, ctr1, params['sa1'], 8, 64)

    nxyz2 = _fps(nxyz1, 128)
    new2 = jnp.transpose(nxyz2, (1, 2, 0))
    gidx2 = _bq(jnp.transpose(nxyz1, (1, 0, 2)), new2, 0.04)
    table2 = jnp.concatenate(
        [new1, l1p.reshape(b, 512, 128), jnp.zeros((b, 512, 5), jnp.float32)],
        axis=-1).reshape(b * 512, 136)
    x02 = _sc_gather(table2, gidx2.reshape(-1, 128), 136)
    ctr2 = jnp.concatenate(
        [new2, jnp.zeros((b, 128, 133), jnp.float32)], axis=-1).reshape(b * 128, 136)
    l2p = _sa_level(x02, ctr2, params['sa2'], 136, 64)

    rows3 = jnp.concatenate(
        [new2, l2p.reshape(b, 128, 256), jnp.zeros((b, 128, 5), jnp.float32)],
        axis=-1).reshape(b * 128, 264)
    sa3 = params['sa3']
    dec = params['decoder']
    args = []
    for l, rows in zip(sa3, (264, 256, 256)):
        args += [_pad_w(l['W'], rows), l['b'][None, :],
                 l['gamma'][None, :], l['beta'][None, :]]
    for l in dec:
        args += [l['W'], l['b'][None, :]]
    out8 = _sa3dec(rows3, args)

    return (out8.reshape(b, 256, 3),
            jnp.transpose(new2, (0, 2, 1)),
            jnp.zeros((b, 3, 1), jnp.float32))
